# Initial kernel scaffold; baseline (speedup 1.0000x reference)
import jax
import jax.numpy as jnp
from jax.experimental import pallas as pl


def kernel(one_hot, features, gemme_features, a_res_indices, a_res_values, W1, b1, W2, b2, W3, b3, W4, b4, cw1, cb1, cw2, cb2, cw3, cb3):
    raise NotImplementedError("write your pallas kernel here")


# speedup vs baseline: 11.8654x; 11.8654x over previous
import functools

import jax
import jax.numpy as jnp
from jax import lax
from jax.experimental import pallas as pl
from jax.experimental.pallas import tpu as pltpu, tpu_sc as plsc

_N = 10000
_E = 160000
_C = 4
_D = 16
_G = _C * _D
_K = 128
_NC, _NS = 2, 16
_NW = _NC * _NS
_NCHUNK = _E // _K
_STEPS = -(-_NCHUNK // _NW)
_RPT = 624
_TAIL = _N - _NS * _RPT
_BM = 1000


def _edge_body(y_hbm, src_hbm, dst_hbm, ewt_hbm, zero_hbm, out_hbm,
               srcidx_v, dstidx_v, ew_v, yg_v, z_v, h_sh, sem):
    cid = lax.axis_index("c")
    sid = lax.axis_index("s")
    wid = sid * _NC + cid

    rows = pl.ds(sid * _RPT, _RPT)
    tail = pl.ds(_NS * _RPT, _TAIL)
    pltpu.sync_copy(zero_hbm.at[rows], h_sh.at[rows])

    @pl.when(sid == _NS - 1)
    def _():
        pltpu.sync_copy(zero_hbm.at[tail], h_sh.at[tail])

    plsc.subcore_barrier()

    def step(it, carry):
        chunk = wid + it * _NW

        @pl.when(chunk < _NCHUNK)
        def _():
            base = chunk * _K
            pltpu.sync_copy(src_hbm.at[pl.ds(base, _K)], srcidx_v)
            pltpu.sync_copy(dst_hbm.at[pl.ds(base, _K)], dstidx_v)
            pltpu.sync_copy(ewt_hbm.at[pl.ds(base * _C, _K * _C)], ew_v)
            pltpu.async_copy(y_hbm.at[srcidx_v], yg_v, sem).wait()
            for p in range(_K // 4):
                ew16 = ew_v[pl.ds(p * 16, 16)]
                for j in range(4):
                    e = p * 4 + j
                    acc = None
                    for c in range(_C):
                        t = ew16[j * _C + c] * yg_v[e, pl.ds(c * _D, 16)]
                        acc = t if acc is None else acc + t
                    z_v[e, :] = acc
            pltpu.sync_copy(z_v, h_sh.at[dstidx_v], add=True)

        return carry

    lax.fori_loop(0, _STEPS, step, 0)
    plsc.subcore_barrier()
    pltpu.sync_copy(h_sh.at[rows],
                    out_hbm.at[pl.ds(cid * _N + sid * _RPT, _RPT)])

    @pl.when(sid == _NS - 1)
    def _():
        pltpu.sync_copy(h_sh.at[tail],
                        out_hbm.at[pl.ds(cid * _N + _NS * _RPT, _TAIL)])


_edge_pass = pl.kernel(
    _edge_body,
    out_type=jax.ShapeDtypeStruct((2 * _N, _D), jnp.float32),
    mesh=plsc.VectorSubcoreMesh(core_axis_name="c", subcore_axis_name="s"),
    compiler_params=pltpu.CompilerParams(use_tc_tiling_on_sc=False),
    scratch_types=[
        pltpu.VMEM((_K,), jnp.int32),
        pltpu.VMEM((_K,), jnp.int32),
        pltpu.VMEM((_K * _C,), jnp.float32),
        pltpu.VMEM((_K, _G), jnp.float32),
        pltpu.VMEM((_K, _D), jnp.float32),
        pltpu.VMEM_SHARED((_N, _D), jnp.float32),
        pltpu.SemaphoreType.DMA,
    ],
)


def _proj_in_body(oh_ref, ft_ref, woh_ref, wft_ref, y_ref):
    y_ref[...] = (
        jnp.dot(oh_ref[...], woh_ref[...], preferred_element_type=jnp.float32)
        + jnp.dot(ft_ref[...], wft_ref[...], preferred_element_type=jnp.float32))


_proj_in = pl.pallas_call(
    _proj_in_body,
    grid=(_N // _BM,),
    in_specs=[
        pl.BlockSpec((_BM, 20), lambda i: (i, 0)),
        pl.BlockSpec((_BM, 236), lambda i: (i, 0)),
        pl.BlockSpec((20, _G), lambda i: (0, 0)),
        pl.BlockSpec((236, _G), lambda i: (0, 0)),
    ],
    out_specs=pl.BlockSpec((_BM, _G), lambda i: (i, 0)),
    out_shape=jax.ShapeDtypeStruct((_N, _G), jnp.float32),
)


def _trans_body(h2_ref, b_ref, w_ref, y_ref):
    h = jnp.maximum(h2_ref[0] + h2_ref[1] + b_ref[...], 0.0)
    y_ref[...] = jnp.dot(h, w_ref[...], preferred_element_type=jnp.float32)


_trans = pl.pallas_call(
    _trans_body,
    grid=(_N // _BM,),
    in_specs=[
        pl.BlockSpec((2, _BM, _D), lambda i: (0, i, 0)),
        pl.BlockSpec((1, _D), lambda i: (0, 0)),
        pl.BlockSpec((_D, _G), lambda i: (0, 0)),
    ],
    out_specs=pl.BlockSpec((_BM, _G), lambda i: (i, 0)),
    out_shape=jax.ShapeDtypeStruct((_N, _G), jnp.float32),
)


def _final_body(h2_ref, b_ref, g2_ref, bias_ref, o_ref):
    i = pl.program_id(0)
    h = jnp.maximum(h2_ref[0] + h2_ref[1] + b_ref[...], 0.0)
    part = jnp.sum(h * g2_ref[...])

    @pl.when(i == 0)
    def _():
        o_ref[0, 0] = bias_ref[0, 0] + part

    @pl.when(i > 0)
    def _():
        o_ref[0, 0] += part


_final = pl.pallas_call(
    _final_body,
    grid=(_N // _BM,),
    in_specs=[
        pl.BlockSpec((2, _BM, _D), lambda i: (0, i, 0)),
        pl.BlockSpec((1, _D), lambda i: (0, 0)),
        pl.BlockSpec((_BM, _D), lambda i: (i, 0)),
        pl.BlockSpec((1, 1), lambda i: (0, 0)),
    ],
    out_specs=pl.BlockSpec((1, 1), lambda i: (0, 0), memory_space=pltpu.SMEM),
    out_shape=jax.ShapeDtypeStruct((1, 1), jnp.float32),
)


def _stack_w(W, rin, rout):
    Wp = jnp.zeros((_C, _D, _D), W.dtype).at[:, :rin, :rout].set(W)
    return Wp.transpose(1, 0, 2).reshape(_D, _G)


def _pad_b(b):
    return jnp.zeros((1, _D), b.dtype).at[0, : b.shape[0]].set(b)


def kernel(one_hot, features, gemme_features, a_res_indices, a_res_values,
           W1, b1, W2, b2, W3, b3, W4, b4, cw1, cb1, cw2, cb2, cw3, cb3):
    src = a_res_indices[0]
    dst = a_res_indices[1]
    ewt = a_res_values.T.reshape(-1)
    zero = jnp.zeros((_N, _D), jnp.float32)

    w1s = W1.transpose(1, 0, 2).reshape(20 + 236, _G)

    y = _proj_in(one_hot, features, w1s[:20], w1s[20:])
    h2 = _edge_pass(y, src, dst, ewt, zero).reshape(2, _N, _D)

    y = _trans(h2, _pad_b(b1), _stack_w(W2, 16, 8))
    h2 = _edge_pass(y, src, dst, ewt, zero).reshape(2, _N, _D)

    y = _trans(h2, _pad_b(b2), _stack_w(W3, 8, 4))
    h2 = _edge_pass(y, src, dst, ewt, zero).reshape(2, _N, _D)

    y = _trans(h2, _pad_b(b3), _stack_w(W4, 4, 1))
    h2 = _edge_pass(y, src, dst, ewt, zero).reshape(2, _N, _D)

    w9, w5, w3 = cw1[0, 0], cw2[0, 0], cw3[0, 0]
    w15 = jnp.convolve(jnp.convolve(w9, w5), w3)
    L = _N - w15.shape[0] + 1
    g = jnp.convolve(jnp.ones((L,), jnp.float32), w15) / L
    g2 = jnp.zeros((_N, _D), jnp.float32).at[:, 0].set(g)
    bias = ((cb1[0] * jnp.sum(w5) + cb2[0]) * jnp.sum(w3) + cb3[0])
    bias = jnp.full((1, 1), 1.0, jnp.float32) * bias

    out = _final(h2, _pad_b(b4), g2, bias)
    return out[0, 0]
